# baseline (speedup 1.0000x reference)
import functools

import jax
import jax.numpy as jnp
from jax import lax
from jax.experimental import pallas as pl
from jax.experimental.pallas import tpu as pltpu
from jax.experimental.pallas import tpu_sc as plsc

_BATCH = 4096
_SEQ = 200
_EMB = 64
_D1 = 16
_VOC = 1000000
_NC = 2
_NS = 16
_NW = _NC * _NS
_RPW = _BATCH // _NW
_C0 = 104
_C1 = _SEQ - _C0
_W = 2 * _EMB

_LANES = 16
_NCH = _EMB // _LANES

_SLAB = 256
_SPW = 126
_NFULL = _VOC // _SLAB
_TAIL = _VOC - _NFULL * _SLAB


def _tr_body(tt_hbm, tail_hbm, out_hbm, slab_a, slab_b, obuf_a, obuf_b,
             tail_v, sem_a, sem_b, osem_a, osem_b):
    wid = lax.axis_index("s") * _NC + lax.axis_index("c")
    slabs = (slab_a, slab_b)
    obufs = (obuf_a, obuf_b)
    sems = (sem_a, sem_b)
    osems = (osem_a, osem_b)
    s0 = wid * _SPW
    nfull = jnp.where(wid < _NW - 1, _SPW, _NFULL - (_NW - 1) * _SPW)

    def stage(s, buf, sem):
        col = pl.multiple_of(s * _SLAB, 128)
        pltpu.async_copy(tt_hbm.at[:, pl.ds(col, _SLAB)], buf, sem)

    def stage_wait(buf, sem):
        pltpu.make_async_copy(tt_hbm.at[:, pl.ds(0, _SLAB)], buf,
                              sem).wait()

    def transpose(s, slab, obuf, osem, n_out):
        lanes = lax.iota(jnp.int32, _LANES)

        def row_body(kl, carry):
            for half in range(2):
                jloc = kl * 2 + half
                jv = jnp.full((_LANES,), 0, jnp.int32) + jloc
                for c in range(_NCH):
                    ev = lanes + (c * _LANES)
                    vals = plsc.load_gather(slab, [ev, jv])
                    obuf[kl, pl.ds(half * _EMB + c * _LANES, _LANES)] = \
                        vals
            return carry

        lax.fori_loop(0, n_out, row_body, 0)
        row0 = pl.multiple_of(s * (_SLAB // 2), 8)
        pltpu.async_copy(obuf.at[pl.ds(0, n_out), :],
                         out_hbm.at[pl.ds(row0, n_out), :], osem)

    def owait(obuf, osem, n_out):
        pltpu.make_async_copy(obuf.at[pl.ds(0, n_out), :],
                              out_hbm.at[pl.ds(0, n_out), :], osem).wait()

    @pl.when(nfull > 0)
    def _():
        stage(s0, slabs[0], sems[0])

    def pair(p, carry):
        for par in (0, 1):
            sl = p * 2 + par
            s = s0 + sl

            @pl.when(sl + 1 < nfull)
            def _():
                stage(s + 1, slabs[1 - par], sems[1 - par])

            @pl.when(sl < nfull)
            def _():
                stage_wait(slabs[par], sems[par])
                @pl.when(sl >= 2)
                def _():
                    owait(obufs[par], osems[par], _SLAB // 2)
                transpose(s, slabs[par], obufs[par], osems[par],
                          _SLAB // 2)
        return carry

    lax.fori_loop(0, (_SPW + 1) // 2, pair, 0)

    @pl.when(nfull >= 2)
    def _():
        owait(obufs[0], osems[0], _SLAB // 2)

    @pl.when(nfull >= 1)
    def _():
        owait(obufs[1], osems[1], _SLAB // 2)

    @pl.when(wid == _NW - 1)
    def _():
        pltpu.sync_copy(tail_hbm, tail_v)

        def tail_row(kl, carry):
            for half in range(2):
                for c in range(_NCH):
                    obuf_a[kl, pl.ds(half * _EMB + c * _LANES, _LANES)] = \
                        tail_v[kl * 2 + half, pl.ds(c * _LANES, _LANES)]
            return carry

        lax.fori_loop(0, _TAIL // 2, tail_row, 0)
        row0 = _NFULL * (_SLAB // 2)
        pltpu.async_copy(obuf_a.at[pl.ds(0, _TAIL // 2), :],
                         out_hbm.at[pl.ds(row0, _TAIL // 2), :], osem_a)
        owait(obuf_a, osem_a, _TAIL // 2)


def _transpose_table(table_t, table_tail):
    mesh = plsc.VectorSubcoreMesh(core_axis_name="c", subcore_axis_name="s")
    f = pl.kernel(
        _tr_body,
        out_type=jax.ShapeDtypeStruct((_VOC // 2, _W), jnp.float32),
        mesh=mesh,
        scratch_types=[
            pltpu.VMEM((_EMB, _SLAB), jnp.float32),
            pltpu.VMEM((_EMB, _SLAB), jnp.float32),
            pltpu.VMEM((_SLAB // 2, _W), jnp.float32),
            pltpu.VMEM((_SLAB // 2, _W), jnp.float32),
            pltpu.VMEM((_TAIL, _EMB), jnp.float32),
            pltpu.SemaphoreType.DMA,
            pltpu.SemaphoreType.DMA,
            pltpu.SemaphoreType.DMA,
            pltpu.SemaphoreType.DMA,
        ],
        compiler_params=pltpu.CompilerParams(needs_layout_passes=False),
    )
    return f(table_t, table_tail)


def _pool_body(idx_hbm, po_hbm, table_hbm, out_hbm, idx_v, po_v,
               rows_a, rows_b, out_v, sem_a, sem_b):
    wid = lax.axis_index("s") * _NC + lax.axis_index("c")
    base = wid * _RPW
    pltpu.sync_copy(idx_hbm.at[pl.ds(wid * (_RPW * _SEQ), _RPW * _SEQ)],
                    idx_v)
    pltpu.sync_copy(po_hbm.at[pl.ds(wid * (_RPW * _SEQ), _RPW * _SEQ)],
                    po_v.at[pl.ds(0, _RPW * _SEQ)])

    bufs = (rows_a, rows_b)
    sems = (sem_a, sem_b)

    def issue(r, buf, sem):
        off = pl.multiple_of(r * _SEQ, 8)
        pltpu.async_copy(
            table_hbm.at[idx_v.at[pl.ds(off, _C0)]],
            buf.at[pl.ds(0, _C0), :], sem)
        pltpu.async_copy(
            table_hbm.at[idx_v.at[pl.ds(off + _C0, _C1)]],
            buf.at[pl.ds(_C0, _C1), :], sem)

    def drain(r, buf, sem):
        off = pl.multiple_of(r * _SEQ, 8)
        pltpu.make_async_copy(
            table_hbm.at[idx_v.at[pl.ds(off, _C0)]],
            buf.at[pl.ds(0, _C0), :], sem).wait()
        pltpu.make_async_copy(
            table_hbm.at[idx_v.at[pl.ds(off + _C0, _C1)]],
            buf.at[pl.ds(_C0, _C1), :], sem).wait()

    def consume(r, buf):
        off = r * _SEQ

        def acc_body(g, carry):
            a, b = carry
            j = g * 8
            po16 = po_v[pl.ds(pl.multiple_of(off + j, 8), _LANES)]
            for k in range(8):
                p = pl.multiple_of(po16[k], 8)
                src = tuple(
                    buf[j + k, pl.ds(p + c * _LANES, _LANES)]
                    for c in range(_NCH))
                if k % 2 == 0:
                    a = tuple(a[c] + src[c] for c in range(_NCH))
                else:
                    b = tuple(b[c] + src[c] for c in range(_NCH))
            return a, b

        zeros = tuple(jnp.zeros((_LANES,), jnp.float32)
                      for _ in range(_NCH))
        a, b = lax.fori_loop(0, _SEQ // 8, acc_body, (zeros, zeros))
        for c in range(_NCH):
            out_v[r, pl.ds(c * _LANES, _LANES)] = \
                (a[c] + b[c]) * (1.0 / _SEQ)

    issue(0, bufs[0], sems[0])

    def pair(p, carry):
        for par in (0, 1):
            r = p * 2 + par
            nxt = r + 1

            @pl.when(nxt < _RPW)
            def _():
                issue(nxt, bufs[1 - par], sems[1 - par])

            drain(r, bufs[par], sems[par])
            consume(r, bufs[par])
        return carry

    lax.fori_loop(0, _RPW // 2, pair, 0)
    pltpu.sync_copy(out_v, out_hbm.at[pl.ds(base, _RPW), :])


def _pool(idx_half, par_off, table2):
    mesh = plsc.VectorSubcoreMesh(core_axis_name="c", subcore_axis_name="s")
    f = pl.kernel(
        _pool_body,
        out_type=jax.ShapeDtypeStruct((_BATCH, _EMB), jnp.float32),
        mesh=mesh,
        scratch_types=[
            pltpu.VMEM((_RPW * _SEQ,), jnp.int32),
            pltpu.VMEM((_RPW * _SEQ + _LANES,), jnp.int32),
            pltpu.VMEM((_SEQ, _W), jnp.float32),
            pltpu.VMEM((_SEQ, _W), jnp.float32),
            pltpu.VMEM((_RPW, _EMB), jnp.float32),
            pltpu.SemaphoreType.DMA,
            pltpu.SemaphoreType.DMA,
        ],
    )
    return f(idx_half, par_off, table2)


def _mlp_body(pooled_ref, w1_ref, b1_ref, w2_ref, b2_ref, out_ref):
    h = jnp.dot(pooled_ref[...], w1_ref[...],
                preferred_element_type=jnp.float32) + b1_ref[...]
    h = jnp.maximum(h, 0.0)
    z = jnp.dot(h, w2_ref[...], preferred_element_type=jnp.float32)
    z = z + b2_ref[...]
    out_ref[...] = 1.0 / (1.0 + jnp.exp(-z))


def kernel(inputs, emb_table, W1, b1, W2, b2):
    idx = inputs.astype(jnp.int32).reshape(-1)
    idx_half = idx // 2
    par_off = (idx & 1) * _EMB
    table2 = _transpose_table(emb_table.T,
                              emb_table[_NFULL * _SLAB:, :])
    pooled = _pool(idx_half, par_off, table2)
    out = pl.pallas_call(
        _mlp_body,
        out_shape=jax.ShapeDtypeStruct((_BATCH, 1), jnp.float32),
    )(pooled, W1, b1.reshape(1, _D1), W2, b2.reshape(1, 1))
    return out


# speedup vs baseline: 1.0000x; 1.0000x over previous
import functools

import jax
import jax.numpy as jnp
from jax import lax
from jax.experimental import pallas as pl
from jax.experimental.pallas import tpu as pltpu
from jax.experimental.pallas import tpu_sc as plsc

_BATCH = 4096
_SEQ = 200
_EMB = 64
_D1 = 16
_VOC = 1000000
_NC = 2
_NS = 16
_NW = _NC * _NS
_RPW = _BATCH // _NW
_C0 = 104
_C1 = _SEQ - _C0
_W = 2 * _EMB

_LANES = 16
_NCH = _EMB // _LANES

_SLAB = 256
_SPW = 126
_NFULL = _VOC // _SLAB
_TAIL = _VOC - _NFULL * _SLAB


def _tr_body(tt_hbm, tail_hbm, out_hbm, slab_a, slab_b, obuf_a, obuf_b,
             tail_v, sem_a, sem_b, osem_a, osem_b):
    wid = lax.axis_index("s") * _NC + lax.axis_index("c")
    slabs = (slab_a, slab_b)
    obufs = (obuf_a, obuf_b)
    sems = (sem_a, sem_b)
    osems = (osem_a, osem_b)
    s0 = wid * _SPW
    nfull = jnp.where(wid < _NW - 1, _SPW, _NFULL - (_NW - 1) * _SPW)

    def stage(s, buf, sem):
        col = pl.multiple_of(s * _SLAB, 128)
        pltpu.async_copy(tt_hbm.at[:, pl.ds(col, _SLAB)], buf, sem)

    def stage_wait(buf, sem):
        pltpu.make_async_copy(tt_hbm.at[:, pl.ds(0, _SLAB)], buf,
                              sem).wait()

    lanes = lax.iota(jnp.int32, _LANES)
    evs = tuple(lanes + c * _LANES for c in range(_NCH))
    zero16 = jnp.zeros((_LANES,), jnp.int32)

    def transpose(s, slab, obuf, osem, n_out):
        def row_body(t, carry):
            for u in range(2):
                kl = t * 2 + u
                for half in range(2):
                    jv = zero16 + (kl * 2 + half)
                    for c in range(_NCH):
                        vals = plsc.load_gather(slab, [evs[c], jv])
                        obuf[kl,
                             pl.ds(half * _EMB + c * _LANES, _LANES)] = \
                            vals
            return carry

        lax.fori_loop(0, n_out // 2, row_body, 0)
        row0 = pl.multiple_of(s * (_SLAB // 2), 8)
        pltpu.async_copy(obuf.at[pl.ds(0, n_out), :],
                         out_hbm.at[pl.ds(row0, n_out), :], osem)

    def owait(obuf, osem, n_out):
        pltpu.make_async_copy(obuf.at[pl.ds(0, n_out), :],
                              out_hbm.at[pl.ds(0, n_out), :], osem).wait()

    @pl.when(nfull > 0)
    def _():
        stage(s0, slabs[0], sems[0])

    def pair(p, carry):
        for par in (0, 1):
            sl = p * 2 + par
            s = s0 + sl

            @pl.when(sl + 1 < nfull)
            def _():
                stage(s + 1, slabs[1 - par], sems[1 - par])

            @pl.when(sl < nfull)
            def _():
                stage_wait(slabs[par], sems[par])
                @pl.when(sl >= 2)
                def _():
                    owait(obufs[par], osems[par], _SLAB // 2)
                transpose(s, slabs[par], obufs[par], osems[par],
                          _SLAB // 2)
        return carry

    lax.fori_loop(0, (_SPW + 1) // 2, pair, 0)

    @pl.when(nfull >= 2)
    def _():
        owait(obufs[0], osems[0], _SLAB // 2)

    @pl.when(nfull >= 1)
    def _():
        owait(obufs[1], osems[1], _SLAB // 2)

    @pl.when(wid == _NW - 1)
    def _():
        pltpu.sync_copy(tail_hbm, tail_v)

        def tail_row(kl, carry):
            for half in range(2):
                for c in range(_NCH):
                    obuf_a[kl, pl.ds(half * _EMB + c * _LANES, _LANES)] = \
                        tail_v[kl * 2 + half, pl.ds(c * _LANES, _LANES)]
            return carry

        lax.fori_loop(0, _TAIL // 2, tail_row, 0)
        row0 = _NFULL * (_SLAB // 2)
        pltpu.async_copy(obuf_a.at[pl.ds(0, _TAIL // 2), :],
                         out_hbm.at[pl.ds(row0, _TAIL // 2), :], osem_a)
        owait(obuf_a, osem_a, _TAIL // 2)


def _transpose_table(table_t, table_tail):
    mesh = plsc.VectorSubcoreMesh(core_axis_name="c", subcore_axis_name="s")
    f = pl.kernel(
        _tr_body,
        out_type=jax.ShapeDtypeStruct((_VOC // 2, _W), jnp.float32),
        mesh=mesh,
        scratch_types=[
            pltpu.VMEM((_EMB, _SLAB), jnp.float32),
            pltpu.VMEM((_EMB, _SLAB), jnp.float32),
            pltpu.VMEM((_SLAB // 2, _W), jnp.float32),
            pltpu.VMEM((_SLAB // 2, _W), jnp.float32),
            pltpu.VMEM((_TAIL, _EMB), jnp.float32),
            pltpu.SemaphoreType.DMA,
            pltpu.SemaphoreType.DMA,
            pltpu.SemaphoreType.DMA,
            pltpu.SemaphoreType.DMA,
        ],
        compiler_params=pltpu.CompilerParams(needs_layout_passes=False),
    )
    return f(table_t, table_tail)


def _pool_body(idx_hbm, po_hbm, table_hbm, out_hbm, idx_v, po_v,
               rows_a, rows_b, out_v, sem_a, sem_b):
    wid = lax.axis_index("s") * _NC + lax.axis_index("c")
    base = wid * _RPW
    pltpu.sync_copy(idx_hbm.at[pl.ds(wid * (_RPW * _SEQ), _RPW * _SEQ)],
                    idx_v)
    pltpu.sync_copy(po_hbm.at[pl.ds(wid * (_RPW * _SEQ), _RPW * _SEQ)],
                    po_v.at[pl.ds(0, _RPW * _SEQ)])

    bufs = (rows_a, rows_b)
    sems = (sem_a, sem_b)

    def issue(r, buf, sem):
        off = pl.multiple_of(r * _SEQ, 8)
        pltpu.async_copy(
            table_hbm.at[idx_v.at[pl.ds(off, _C0)]],
            buf.at[pl.ds(0, _C0), :], sem)
        pltpu.async_copy(
            table_hbm.at[idx_v.at[pl.ds(off + _C0, _C1)]],
            buf.at[pl.ds(_C0, _C1), :], sem)

    def drain(r, buf, sem):
        off = pl.multiple_of(r * _SEQ, 8)
        pltpu.make_async_copy(
            table_hbm.at[idx_v.at[pl.ds(off, _C0)]],
            buf.at[pl.ds(0, _C0), :], sem).wait()
        pltpu.make_async_copy(
            table_hbm.at[idx_v.at[pl.ds(off + _C0, _C1)]],
            buf.at[pl.ds(_C0, _C1), :], sem).wait()

    def consume(r, buf):
        off = r * _SEQ

        def acc_body(g, carry):
            a, b = carry
            j = g * 8
            po16 = po_v[pl.ds(pl.multiple_of(off + j, 8), _LANES)]
            for k in range(8):
                p = pl.multiple_of(po16[k], 8)
                src = tuple(
                    buf[j + k, pl.ds(p + c * _LANES, _LANES)]
                    for c in range(_NCH))
                if k % 2 == 0:
                    a = tuple(a[c] + src[c] for c in range(_NCH))
                else:
                    b = tuple(b[c] + src[c] for c in range(_NCH))
            return a, b

        zeros = tuple(jnp.zeros((_LANES,), jnp.float32)
                      for _ in range(_NCH))
        a, b = lax.fori_loop(0, _SEQ // 8, acc_body, (zeros, zeros))
        for c in range(_NCH):
            out_v[r, pl.ds(c * _LANES, _LANES)] = \
                (a[c] + b[c]) * (1.0 / _SEQ)

    issue(0, bufs[0], sems[0])

    def pair(p, carry):
        for par in (0, 1):
            r = p * 2 + par
            nxt = r + 1

            @pl.when(nxt < _RPW)
            def _():
                issue(nxt, bufs[1 - par], sems[1 - par])

            drain(r, bufs[par], sems[par])
            consume(r, bufs[par])
        return carry

    lax.fori_loop(0, _RPW // 2, pair, 0)
    pltpu.sync_copy(out_v, out_hbm.at[pl.ds(base, _RPW), :])


def _pool(idx_half, par_off, table2):
    mesh = plsc.VectorSubcoreMesh(core_axis_name="c", subcore_axis_name="s")
    f = pl.kernel(
        _pool_body,
        out_type=jax.ShapeDtypeStruct((_BATCH, _EMB), jnp.float32),
        mesh=mesh,
        scratch_types=[
            pltpu.VMEM((_RPW * _SEQ,), jnp.int32),
            pltpu.VMEM((_RPW * _SEQ + _LANES,), jnp.int32),
            pltpu.VMEM((_SEQ, _W), jnp.float32),
            pltpu.VMEM((_SEQ, _W), jnp.float32),
            pltpu.VMEM((_RPW, _EMB), jnp.float32),
            pltpu.SemaphoreType.DMA,
            pltpu.SemaphoreType.DMA,
        ],
    )
    return f(idx_half, par_off, table2)


def _mlp_body(pooled_ref, w1_ref, b1_ref, w2_ref, b2_ref, out_ref):
    h = jnp.dot(pooled_ref[...], w1_ref[...],
                preferred_element_type=jnp.float32) + b1_ref[...]
    h = jnp.maximum(h, 0.0)
    z = jnp.dot(h, w2_ref[...], preferred_element_type=jnp.float32)
    z = z + b2_ref[...]
    out_ref[...] = 1.0 / (1.0 + jnp.exp(-z))


def kernel(inputs, emb_table, W1, b1, W2, b2):
    idx = inputs.astype(jnp.int32).reshape(-1)
    idx_half = idx // 2
    par_off = (idx & 1) * _EMB
    table2 = _transpose_table(emb_table.T,
                              emb_table[_NFULL * _SLAB:, :])
    pooled = _pool(idx_half, par_off, table2)
    out = pl.pallas_call(
        _mlp_body,
        out_shape=jax.ShapeDtypeStruct((_BATCH, 1), jnp.float32),
    )(pooled, W1, b1.reshape(1, _D1), W2, b2.reshape(1, 1))
    return out


# speedup vs baseline: 2.3955x; 2.3954x over previous
import functools

import jax
import jax.numpy as jnp
from jax import lax
from jax.experimental import pallas as pl
from jax.experimental.pallas import tpu as pltpu
from jax.experimental.pallas import tpu_sc as plsc

_BATCH = 4096
_SEQ = 200
_EMB = 64
_D1 = 16
_NC = 2
_NS = 16
_NW = _NC * _NS
_RPW = _BATCH // _NW
_C0 = 104
_C1 = _SEQ - _C0

_LANES = 16
_NCH = _EMB // _LANES


def _pool_body(idx_hbm, table_hbm, out_hbm, idx_v, rows_a, rows_b,
               out_v, sem_a, sem_b):
    wid = lax.axis_index("s") * _NC + lax.axis_index("c")
    base = wid * _RPW
    pltpu.sync_copy(idx_hbm.at[pl.ds(wid * (_RPW * _SEQ), _RPW * _SEQ)],
                    idx_v)

    bufs = (rows_a, rows_b)
    sems = (sem_a, sem_b)

    def issue(r, buf, sem):
        off = pl.multiple_of(r * _SEQ, 8)
        pltpu.async_copy(
            table_hbm.at[idx_v.at[pl.ds(off, _C0)]],
            buf.at[pl.ds(0, _C0), :], sem)
        pltpu.async_copy(
            table_hbm.at[idx_v.at[pl.ds(off + _C0, _C1)]],
            buf.at[pl.ds(_C0, _C1), :], sem)

    def drain(r, buf, sem):
        off = pl.multiple_of(r * _SEQ, 8)
        pltpu.make_async_copy(
            table_hbm.at[idx_v.at[pl.ds(off, _C0)]],
            buf.at[pl.ds(0, _C0), :], sem).wait()
        pltpu.make_async_copy(
            table_hbm.at[idx_v.at[pl.ds(off + _C0, _C1)]],
            buf.at[pl.ds(_C0, _C1), :], sem).wait()

    def consume(r, buf):
        def acc_body(t, carry):
            a, b = carry
            j = t * 4
            for q in range(4):
                src = tuple(
                    buf[j + q, pl.ds(c * _LANES, _LANES)]
                    for c in range(_NCH))
                if q % 2 == 0:
                    a = tuple(a[c] + src[c] for c in range(_NCH))
                else:
                    b = tuple(b[c] + src[c] for c in range(_NCH))
            return a, b

        zeros = tuple(jnp.zeros((_LANES,), jnp.float32)
                      for _ in range(_NCH))
        a, b = lax.fori_loop(0, _SEQ // 4, acc_body, (zeros, zeros))
        for c in range(_NCH):
            out_v[r, pl.ds(c * _LANES, _LANES)] = \
                (a[c] + b[c]) * (1.0 / _SEQ)

    issue(0, bufs[0], sems[0])

    def pair(p, carry):
        for par in (0, 1):
            r = p * 2 + par
            nxt = r + 1

            @pl.when(nxt < _RPW)
            def _():
                issue(nxt, bufs[1 - par], sems[1 - par])

            drain(r, bufs[par], sems[par])
            consume(r, bufs[par])
        return carry

    lax.fori_loop(0, _RPW // 2, pair, 0)
    pltpu.sync_copy(out_v, out_hbm.at[pl.ds(base, _RPW), :])


def _pool(idx, table):
    mesh = plsc.VectorSubcoreMesh(core_axis_name="c", subcore_axis_name="s")
    f = pl.kernel(
        _pool_body,
        out_type=jax.ShapeDtypeStruct((_BATCH, _EMB), jnp.float32),
        mesh=mesh,
        scratch_types=[
            pltpu.VMEM((_RPW * _SEQ,), jnp.int32),
            pltpu.VMEM((_SEQ, _EMB), jnp.float32),
            pltpu.VMEM((_SEQ, _EMB), jnp.float32),
            pltpu.VMEM((_RPW, _EMB), jnp.float32),
            pltpu.SemaphoreType.DMA,
            pltpu.SemaphoreType.DMA,
        ],
        compiler_params=pltpu.CompilerParams(use_tc_tiling_on_sc=False),
    )
    return f(idx, table)


def _mlp_body(pooled_ref, w1_ref, b1_ref, w2_ref, b2_ref, out_ref):
    h = jnp.dot(pooled_ref[...], w1_ref[...],
                preferred_element_type=jnp.float32) + b1_ref[...]
    h = jnp.maximum(h, 0.0)
    z = jnp.dot(h, w2_ref[...], preferred_element_type=jnp.float32)
    z = z + b2_ref[...]
    out_ref[...] = 1.0 / (1.0 + jnp.exp(-z))


def kernel(inputs, emb_table, W1, b1, W2, b2):
    idx = inputs.astype(jnp.int32).reshape(-1)
    pooled = _pool(idx, emb_table)
    out = pl.pallas_call(
        _mlp_body,
        out_shape=jax.ShapeDtypeStruct((_BATCH, 1), jnp.float32),
    )(pooled, W1, b1.reshape(1, _D1), W2, b2.reshape(1, 1))
    return out
